# baseline (speedup 1.0000x reference)
import jax
import jax.numpy as jnp
from jax import lax
from jax.experimental import pallas as pl
from jax.experimental.pallas import tpu as pltpu
from jax.experimental.pallas import tpu_sc as plsc

N = 10000
E = 160000
D_IN = 128
LATENT = 64
HID = 128

NC, NS = 2, 16
NW = NC * NS
NP = 10240
EP = 163840
EPT = EP // NW
CHUNK = 128
NCHUNK = EPT // CHUNK
NCH_A = 20
NCH_B = 60
ROWS_PT = NP // NS

_MESH = plsc.VectorSubcoreMesh(
    core_axis_name="c", subcore_axis_name="s", num_cores=NC, num_subcores=NS)


def _worker(cid, sid):
    return sid * NC + cid


def _hist_body(dst2_hbm, ones_hbm, z_hbm, out_hbm, didx_v, ones_v, rows_v,
               acc_sh, sem):
    cid = lax.axis_index("c")
    sid = lax.axis_index("s")
    wid = _worker(cid, sid)
    pltpu.sync_copy(dst2_hbm.at[wid], didx_v)
    pltpu.sync_copy(ones_hbm, ones_v)
    pltpu.sync_copy(z_hbm, rows_v)
    for j in range(ROWS_PT // CHUNK):
        pltpu.sync_copy(rows_v, acc_sh.at[pl.ds(sid * ROWS_PT + j * CHUNK, CHUNK)])
    plsc.subcore_barrier()

    def wave(i, carry):
        k = 8 * i
        for j in range(8):
            pltpu.async_copy(ones_v, acc_sh.at[didx_v.at[k + j]], sem, add=True)
        for j in range(8):
            pltpu.make_async_copy(ones_v, acc_sh.at[didx_v.at[k + j]], sem).wait()
        return carry

    lax.fori_loop(0, NCHUNK // 8, wave, 0)
    plsc.subcore_barrier()
    for j in range(ROWS_PT // CHUNK):
        rows = pl.ds(sid * ROWS_PT + j * CHUNK, CHUNK)
        pltpu.sync_copy(acc_sh.at[rows], rows_v)
        pltpu.sync_copy(rows_v, out_hbm.at[cid, rows])


_hist_call = pl.kernel(
    _hist_body,
    out_type=jax.ShapeDtypeStruct((NC, NP, HID), jnp.float32),
    mesh=_MESH,
    scratch_types=[
        pltpu.VMEM((NCHUNK + 2, CHUNK), jnp.int32),
        pltpu.VMEM((CHUNK, HID), jnp.float32),
        pltpu.VMEM((CHUNK, HID), jnp.float32),
        pltpu.VMEM_SHARED((NP, HID), jnp.float32),
        pltpu.SemaphoreType.DMA,
    ],
)


def _agg_body(y_hbm, src_hbm, dst_hbm, z_hbm, out_hbm,
              sidx0_v, didx0_v, sidx1_v, didx1_v, rows0_v, rows1_v,
              acc_sh, sem0, sem1):
    cid = lax.axis_index("c")
    sid = lax.axis_index("s")
    pltpu.sync_copy(z_hbm, rows0_v)
    for j in range(ROWS_PT // CHUNK):
        pltpu.sync_copy(rows0_v, acc_sh.at[pl.ds(sid * ROWS_PT + j * CHUNK, CHUNK)])
    plsc.subcore_barrier()

    base_chunk = jnp.where(cid == 0, sid * NCH_A, NS * NCH_A + sid * NCH_B)
    nch = jnp.where(cid == 0, NCH_A, NCH_B)

    def pair(i, carry):
        base0 = (base_chunk + 2 * i) * CHUNK
        base1 = base0 + CHUNK
        pltpu.sync_copy(src_hbm.at[pl.ds(base0, CHUNK)], sidx0_v)
        d0 = pltpu.async_copy(y_hbm.at[sidx0_v], rows0_v, sem0)
        pltpu.sync_copy(src_hbm.at[pl.ds(base1, CHUNK)], sidx1_v)
        d1 = pltpu.async_copy(y_hbm.at[sidx1_v], rows1_v, sem1)
        pltpu.sync_copy(dst_hbm.at[pl.ds(base0, CHUNK)], didx0_v)
        pltpu.sync_copy(dst_hbm.at[pl.ds(base1, CHUNK)], didx1_v)
        d0.wait()
        pltpu.sync_copy(rows0_v, acc_sh.at[didx0_v], add=True)
        d1.wait()
        pltpu.sync_copy(rows1_v, acc_sh.at[didx1_v], add=True)
        return carry

    lax.fori_loop(0, nch // 2, pair, 0)
    plsc.subcore_barrier()
    for j in range(ROWS_PT // CHUNK):
        rows = pl.ds(sid * ROWS_PT + j * CHUNK, CHUNK)
        pltpu.sync_copy(acc_sh.at[rows], rows0_v)
        pltpu.sync_copy(rows0_v, out_hbm.at[cid, rows])


_agg128 = pl.kernel(
    _agg_body,
    out_type=jax.ShapeDtypeStruct((NC, NP, HID), jnp.float32),
    mesh=_MESH,
    scratch_types=[
        pltpu.VMEM((CHUNK,), jnp.int32),
        pltpu.VMEM((CHUNK,), jnp.int32),
        pltpu.VMEM((CHUNK,), jnp.int32),
        pltpu.VMEM((CHUNK,), jnp.int32),
        pltpu.VMEM((CHUNK, HID), jnp.float32),
        pltpu.VMEM((CHUNK, HID), jnp.float32),
        pltpu.VMEM_SHARED((NP, HID), jnp.float32),
        pltpu.SemaphoreType.DMA,
        pltpu.SemaphoreType.DMA,
    ],
)


_BR = 256
_GRID = NP // _BR


def _t1a_body(x_ref, w1_ref, xw_ref):
    xw_ref[...] = jnp.dot(x_ref[...], w1_ref[...], preferred_element_type=jnp.float32)


def _t1b_body(hist_ref, xw_ref, y1_ref, dinv_ref):
    deg = hist_ref[0, :, 0:1] + hist_ref[1, :, 0:1] + 2.0
    dinv = lax.rsqrt(deg)
    y1_ref[...] = xw_ref[...] * dinv
    dinv_ref[...] = dinv


def _t2_body(acc_ref, y1_ref, dinv_ref, w2_ref, b1_ref, y2_ref):
    dinv = dinv_ref[...]
    h = dinv * (acc_ref[0] + acc_ref[1] + 2.0 * y1_ref[...]) + b1_ref[...]
    h = jnp.maximum(h, 0.0)
    y2_ref[...] = dinv * jnp.dot(h, w2_ref[...], preferred_element_type=jnp.float32)


def _t3_body(acc_ref, y2_ref, dinv_ref, wm_ref, wl_ref, b2_ref, bm_ref, bl_ref,
             mu_ref, lv_ref):
    h2 = dinv_ref[...] * (acc_ref[0] + acc_ref[1] + 2.0 * y2_ref[...]) + b2_ref[...]
    mu_ref[...] = jnp.dot(h2, wm_ref[...], preferred_element_type=jnp.float32) + bm_ref[...]
    lv_ref[...] = jnp.dot(h2, wl_ref[...], preferred_element_type=jnp.float32) + bl_ref[...]


_DB = 512


def _dec_body(mu_i_ref, mu_j_ref, adj_ref):
    prod = lax.dot_general(mu_i_ref[...], mu_j_ref[...],
                           (((1,), (1,)), ((), ())),
                           preferred_element_type=jnp.float32)
    adj_ref[...] = jax.nn.sigmoid(prod)


def kernel(x, edge_index, W1, b1, W2, b2, Wm, bm, Wl, bl):
    src = edge_index[0]
    dst = edge_index[1]
    pad_e = EP - E
    srcp = jnp.concatenate([src, jnp.zeros((pad_e,), jnp.int32)])
    dstp = jnp.concatenate([dst, jnp.full((pad_e,), N, jnp.int32)])
    src2 = jnp.pad(srcp.reshape(NW, NCHUNK, CHUNK), ((0, 0), (0, 2), (0, 0)))
    dst2 = jnp.pad(dstp.reshape(NW, NCHUNK, CHUNK), ((0, 0), (0, 2), (0, 0)))
    xp = jnp.pad(x, ((0, NP - N), (0, 0)))
    zc = jnp.zeros((CHUNK, HID), jnp.float32)
    W2p = jnp.pad(W2, ((0, 0), (0, HID - LATENT)))
    ones_hot = jnp.zeros((CHUNK, HID), jnp.float32).at[:, 0].set(1.0)

    hist = _hist_call(dst2, ones_hot, zc)

    xw = pl.pallas_call(
        _t1a_body,
        grid=(_GRID,),
        in_specs=[
            pl.BlockSpec((_BR, D_IN), lambda i: (i, 0)),
            pl.BlockSpec((D_IN, HID), lambda i: (0, 0)),
        ],
        out_specs=pl.BlockSpec((_BR, HID), lambda i: (i, 0)),
        out_shape=jax.ShapeDtypeStruct((NP, HID), jnp.float32),
    )(xp, W1)

    y1, dinv = pl.pallas_call(
        _t1b_body,
        grid=(_GRID,),
        in_specs=[
            pl.BlockSpec((NC, _BR, HID), lambda i: (0, i, 0)),
            pl.BlockSpec((_BR, HID), lambda i: (i, 0)),
        ],
        out_specs=[
            pl.BlockSpec((_BR, HID), lambda i: (i, 0)),
            pl.BlockSpec((_BR, 1), lambda i: (i, 0)),
        ],
        out_shape=[
            jax.ShapeDtypeStruct((NP, HID), jnp.float32),
            jax.ShapeDtypeStruct((NP, 1), jnp.float32),
        ],
    )(hist, xw)

    acc1 = _agg128(y1, srcp, dstp, zc)

    y2 = pl.pallas_call(
        _t2_body,
        grid=(_GRID,),
        in_specs=[
            pl.BlockSpec((NC, _BR, HID), lambda i: (0, i, 0)),
            pl.BlockSpec((_BR, HID), lambda i: (i, 0)),
            pl.BlockSpec((_BR, 1), lambda i: (i, 0)),
            pl.BlockSpec((HID, HID), lambda i: (0, 0)),
            pl.BlockSpec((1, HID), lambda i: (0, 0)),
        ],
        out_specs=pl.BlockSpec((_BR, HID), lambda i: (i, 0)),
        out_shape=jax.ShapeDtypeStruct((NP, HID), jnp.float32),
    )(acc1, y1, dinv, W2p, b1.reshape(1, HID))

    acc2 = _agg128(y2, srcp, dstp, zc)

    mu_p, lv_p = pl.pallas_call(
        _t3_body,
        grid=(_GRID,),
        in_specs=[
            pl.BlockSpec((NC, _BR, HID), lambda i: (0, i, 0)),
            pl.BlockSpec((_BR, HID), lambda i: (i, 0)),
            pl.BlockSpec((_BR, 1), lambda i: (i, 0)),
            pl.BlockSpec((HID, LATENT), lambda i: (0, 0)),
            pl.BlockSpec((HID, LATENT), lambda i: (0, 0)),
            pl.BlockSpec((1, HID), lambda i: (0, 0)),
            pl.BlockSpec((1, LATENT), lambda i: (0, 0)),
            pl.BlockSpec((1, LATENT), lambda i: (0, 0)),
        ],
        out_specs=[
            pl.BlockSpec((_BR, LATENT), lambda i: (i, 0)),
            pl.BlockSpec((_BR, LATENT), lambda i: (i, 0)),
        ],
        out_shape=[
            jax.ShapeDtypeStruct((NP, LATENT), jnp.float32),
            jax.ShapeDtypeStruct((NP, LATENT), jnp.float32),
        ],
    )(acc2, y2, dinv,
      jnp.pad(Wm, ((0, HID - LATENT), (0, 0))),
      jnp.pad(Wl, ((0, HID - LATENT), (0, 0))),
      jnp.pad(b2, (0, HID - LATENT)).reshape(1, HID),
      bm.reshape(1, LATENT), bl.reshape(1, LATENT))

    adj = pl.pallas_call(
        _dec_body,
        grid=(N // _DB + 1, N // _DB + 1),
        in_specs=[
            pl.BlockSpec((_DB, LATENT), lambda i, j: (i, 0)),
            pl.BlockSpec((_DB, LATENT), lambda i, j: (j, 0)),
        ],
        out_specs=pl.BlockSpec((_DB, _DB), lambda i, j: (i, j)),
        out_shape=jax.ShapeDtypeStruct((N, N), jnp.float32),
    )(mu_p, mu_p)

    return (adj, mu_p[:N], lv_p[:N])


# speedup vs baseline: 1.1231x; 1.1231x over previous
import jax
import jax.numpy as jnp
from jax import lax
from jax.experimental import pallas as pl
from jax.experimental.pallas import tpu as pltpu
from jax.experimental.pallas import tpu_sc as plsc

N = 10000
E = 160000
D_IN = 128
LATENT = 64
HID = 128

NC, NS = 2, 16
NW = NC * NS
NP = 10240
EP = 163840
EPT = EP // NW
CHUNK = 128
NCHUNK = EPT // CHUNK
NCH_A = 60
NCH_B = 20
ROWS_PT = NP // NS

_MESH = plsc.VectorSubcoreMesh(
    core_axis_name="c", subcore_axis_name="s", num_cores=NC, num_subcores=NS)


def _worker(cid, sid):
    return sid * NC + cid


def _hist_body(dst2_hbm, ones_hbm, z_hbm, out_hbm, didx_v, ones_v, rows_v,
               acc_sh, sem):
    cid = lax.axis_index("c")
    sid = lax.axis_index("s")
    wid = _worker(cid, sid)
    pltpu.sync_copy(dst2_hbm.at[wid], didx_v)
    pltpu.sync_copy(ones_hbm, ones_v)
    pltpu.sync_copy(z_hbm, rows_v)
    for j in range(ROWS_PT // CHUNK):
        pltpu.sync_copy(rows_v, acc_sh.at[pl.ds(sid * ROWS_PT + j * CHUNK, CHUNK)])
    plsc.subcore_barrier()

    def wave(i, carry):
        k = 8 * i
        for j in range(8):
            pltpu.async_copy(ones_v, acc_sh.at[didx_v.at[k + j]], sem, add=True)
        for j in range(8):
            pltpu.make_async_copy(ones_v, acc_sh.at[didx_v.at[k + j]], sem).wait()
        return carry

    lax.fori_loop(0, NCHUNK // 8, wave, 0)
    plsc.subcore_barrier()
    for j in range(ROWS_PT // CHUNK):
        rows = pl.ds(sid * ROWS_PT + j * CHUNK, CHUNK)
        pltpu.sync_copy(acc_sh.at[rows], rows_v)
        pltpu.sync_copy(rows_v, out_hbm.at[cid, rows])


_hist_call = pl.kernel(
    _hist_body,
    out_type=jax.ShapeDtypeStruct((NC, NP, HID), jnp.float32),
    mesh=_MESH,
    scratch_types=[
        pltpu.VMEM((NCHUNK + 2, CHUNK), jnp.int32),
        pltpu.VMEM((CHUNK, HID), jnp.float32),
        pltpu.VMEM((CHUNK, HID), jnp.float32),
        pltpu.VMEM_SHARED((NP, HID), jnp.float32),
        pltpu.SemaphoreType.DMA,
    ],
)


def _agg_body(y_hbm, src_hbm, dst_hbm, z_hbm, out_hbm,
              sidx0_v, didx0_v, sidx1_v, didx1_v, rows0_v, rows1_v,
              acc_sh, sem0, sem1):
    cid = lax.axis_index("c")
    sid = lax.axis_index("s")
    pltpu.sync_copy(z_hbm, rows0_v)
    for j in range(ROWS_PT // CHUNK):
        pltpu.sync_copy(rows0_v, acc_sh.at[pl.ds(sid * ROWS_PT + j * CHUNK, CHUNK)])
    plsc.subcore_barrier()

    base_chunk = jnp.where(cid == 0, sid * NCH_A, NS * NCH_A + sid * NCH_B)
    nch = jnp.where(cid == 0, NCH_A, NCH_B)

    def pair(i, carry):
        base0 = (base_chunk + 2 * i) * CHUNK
        base1 = base0 + CHUNK
        pltpu.sync_copy(src_hbm.at[pl.ds(base0, CHUNK)], sidx0_v)
        d0 = pltpu.async_copy(y_hbm.at[sidx0_v], rows0_v, sem0)
        pltpu.sync_copy(src_hbm.at[pl.ds(base1, CHUNK)], sidx1_v)
        d1 = pltpu.async_copy(y_hbm.at[sidx1_v], rows1_v, sem1)
        pltpu.sync_copy(dst_hbm.at[pl.ds(base0, CHUNK)], didx0_v)
        pltpu.sync_copy(dst_hbm.at[pl.ds(base1, CHUNK)], didx1_v)
        d0.wait()
        pltpu.sync_copy(rows0_v, acc_sh.at[didx0_v], add=True)
        d1.wait()
        pltpu.sync_copy(rows1_v, acc_sh.at[didx1_v], add=True)
        return carry

    lax.fori_loop(0, nch // 2, pair, 0)
    plsc.subcore_barrier()
    for j in range(ROWS_PT // CHUNK):
        rows = pl.ds(sid * ROWS_PT + j * CHUNK, CHUNK)
        pltpu.sync_copy(acc_sh.at[rows], rows0_v)
        pltpu.sync_copy(rows0_v, out_hbm.at[cid, rows])


_agg128 = pl.kernel(
    _agg_body,
    out_type=jax.ShapeDtypeStruct((NC, NP, HID), jnp.float32),
    mesh=_MESH,
    scratch_types=[
        pltpu.VMEM((CHUNK,), jnp.int32),
        pltpu.VMEM((CHUNK,), jnp.int32),
        pltpu.VMEM((CHUNK,), jnp.int32),
        pltpu.VMEM((CHUNK,), jnp.int32),
        pltpu.VMEM((CHUNK, HID), jnp.float32),
        pltpu.VMEM((CHUNK, HID), jnp.float32),
        pltpu.VMEM_SHARED((NP, HID), jnp.float32),
        pltpu.SemaphoreType.DMA,
        pltpu.SemaphoreType.DMA,
    ],
)


_BR = 256
_GRID = NP // _BR


def _t1a_body(x_ref, w1_ref, xw_ref):
    xw_ref[...] = jnp.dot(x_ref[...], w1_ref[...], preferred_element_type=jnp.float32)


def _t1b_body(hist_ref, xw_ref, y1_ref, dinv_ref):
    deg = hist_ref[0, :, 0:1] + hist_ref[1, :, 0:1] + 2.0
    dinv = lax.rsqrt(deg)
    y1_ref[...] = xw_ref[...] * dinv
    dinv_ref[...] = dinv


def _t2_body(acc_ref, y1_ref, dinv_ref, w2_ref, b1_ref, y2_ref):
    dinv = dinv_ref[...]
    h = dinv * (acc_ref[0] + acc_ref[1] + 2.0 * y1_ref[...]) + b1_ref[...]
    h = jnp.maximum(h, 0.0)
    y2_ref[...] = dinv * jnp.dot(h, w2_ref[...], preferred_element_type=jnp.float32)


def _t3_body(acc_ref, y2_ref, dinv_ref, wm_ref, wl_ref, b2_ref, bm_ref, bl_ref,
             mu_ref, lv_ref):
    h2 = dinv_ref[...] * (acc_ref[0] + acc_ref[1] + 2.0 * y2_ref[...]) + b2_ref[...]
    mu_ref[...] = jnp.dot(h2, wm_ref[...], preferred_element_type=jnp.float32) + bm_ref[...]
    lv_ref[...] = jnp.dot(h2, wl_ref[...], preferred_element_type=jnp.float32) + bl_ref[...]


_DB = 512


def _dec_body(mu_i_ref, mu_j_ref, adj_ref):
    prod = lax.dot_general(mu_i_ref[...], mu_j_ref[...],
                           (((1,), (1,)), ((), ())),
                           preferred_element_type=jnp.float32)
    adj_ref[...] = jax.nn.sigmoid(prod)


def kernel(x, edge_index, W1, b1, W2, b2, Wm, bm, Wl, bl):
    src = edge_index[0]
    dst = edge_index[1]
    pad_e = EP - E
    srcp = jnp.concatenate([src, jnp.zeros((pad_e,), jnp.int32)])
    dstp = jnp.concatenate([dst, jnp.full((pad_e,), N, jnp.int32)])
    src2 = jnp.pad(srcp.reshape(NW, NCHUNK, CHUNK), ((0, 0), (0, 2), (0, 0)))
    dst2 = jnp.pad(dstp.reshape(NW, NCHUNK, CHUNK), ((0, 0), (0, 2), (0, 0)))
    xp = jnp.pad(x, ((0, NP - N), (0, 0)))
    zc = jnp.zeros((CHUNK, HID), jnp.float32)
    W2p = jnp.pad(W2, ((0, 0), (0, HID - LATENT)))
    ones_hot = jnp.zeros((CHUNK, HID), jnp.float32).at[:, 0].set(1.0)

    hist = _hist_call(dst2, ones_hot, zc)

    xw = pl.pallas_call(
        _t1a_body,
        grid=(_GRID,),
        in_specs=[
            pl.BlockSpec((_BR, D_IN), lambda i: (i, 0)),
            pl.BlockSpec((D_IN, HID), lambda i: (0, 0)),
        ],
        out_specs=pl.BlockSpec((_BR, HID), lambda i: (i, 0)),
        out_shape=jax.ShapeDtypeStruct((NP, HID), jnp.float32),
    )(xp, W1)

    y1, dinv = pl.pallas_call(
        _t1b_body,
        grid=(_GRID,),
        in_specs=[
            pl.BlockSpec((NC, _BR, HID), lambda i: (0, i, 0)),
            pl.BlockSpec((_BR, HID), lambda i: (i, 0)),
        ],
        out_specs=[
            pl.BlockSpec((_BR, HID), lambda i: (i, 0)),
            pl.BlockSpec((_BR, 1), lambda i: (i, 0)),
        ],
        out_shape=[
            jax.ShapeDtypeStruct((NP, HID), jnp.float32),
            jax.ShapeDtypeStruct((NP, 1), jnp.float32),
        ],
    )(hist, xw)

    acc1 = _agg128(y1, srcp, dstp, zc)

    y2 = pl.pallas_call(
        _t2_body,
        grid=(_GRID,),
        in_specs=[
            pl.BlockSpec((NC, _BR, HID), lambda i: (0, i, 0)),
            pl.BlockSpec((_BR, HID), lambda i: (i, 0)),
            pl.BlockSpec((_BR, 1), lambda i: (i, 0)),
            pl.BlockSpec((HID, HID), lambda i: (0, 0)),
            pl.BlockSpec((1, HID), lambda i: (0, 0)),
        ],
        out_specs=pl.BlockSpec((_BR, HID), lambda i: (i, 0)),
        out_shape=jax.ShapeDtypeStruct((NP, HID), jnp.float32),
    )(acc1, y1, dinv, W2p, b1.reshape(1, HID))

    acc2 = _agg128(y2, srcp, dstp, zc)

    mu_p, lv_p = pl.pallas_call(
        _t3_body,
        grid=(_GRID,),
        in_specs=[
            pl.BlockSpec((NC, _BR, HID), lambda i: (0, i, 0)),
            pl.BlockSpec((_BR, HID), lambda i: (i, 0)),
            pl.BlockSpec((_BR, 1), lambda i: (i, 0)),
            pl.BlockSpec((HID, LATENT), lambda i: (0, 0)),
            pl.BlockSpec((HID, LATENT), lambda i: (0, 0)),
            pl.BlockSpec((1, HID), lambda i: (0, 0)),
            pl.BlockSpec((1, LATENT), lambda i: (0, 0)),
            pl.BlockSpec((1, LATENT), lambda i: (0, 0)),
        ],
        out_specs=[
            pl.BlockSpec((_BR, LATENT), lambda i: (i, 0)),
            pl.BlockSpec((_BR, LATENT), lambda i: (i, 0)),
        ],
        out_shape=[
            jax.ShapeDtypeStruct((NP, LATENT), jnp.float32),
            jax.ShapeDtypeStruct((NP, LATENT), jnp.float32),
        ],
    )(acc2, y2, dinv,
      jnp.pad(Wm, ((0, HID - LATENT), (0, 0))),
      jnp.pad(Wl, ((0, HID - LATENT), (0, 0))),
      jnp.pad(b2, (0, HID - LATENT)).reshape(1, HID),
      bm.reshape(1, LATENT), bl.reshape(1, LATENT))

    adj = pl.pallas_call(
        _dec_body,
        grid=(N // _DB + 1, N // _DB + 1),
        in_specs=[
            pl.BlockSpec((_DB, LATENT), lambda i, j: (i, 0)),
            pl.BlockSpec((_DB, LATENT), lambda i, j: (j, 0)),
        ],
        out_specs=pl.BlockSpec((_DB, _DB), lambda i, j: (i, j)),
        out_shape=jax.ShapeDtypeStruct((N, N), jnp.float32),
    )(mu_p, mu_p)

    return (adj, mu_p[:N], lv_p[:N])
